# baseline (speedup 1.0000x reference)
import functools

import jax
import jax.numpy as jnp
from jax import lax
from jax.experimental import pallas as pl
from jax.experimental.pallas import tpu as pltpu
from jax.experimental.pallas import tpu_sc as plsc

B = 128
R = 2048
C = 128
TOPK_K = 32
INV_N = 1.0 / (R * C)

S_TC = 96
K_SC = B - S_TC
BB = 4

NT = 32
RT = R // NT
L = 16


NBUF = 2


CS = 8


def _mse_manual_body(o_hbm, l_hbm, out_ref, obuf, lbuf, osem, lsem):
    NCH = S_TC // CS

    def _copies(c, slot):
        yield pltpu.make_async_copy(
            o_hbm.at[pl.ds(c * CS, CS), :, :], obuf.at[slot], osem.at[slot])
        yield pltpu.make_async_copy(
            l_hbm.at[pl.ds(c * CS, CS), :, :], lbuf.at[slot], lsem.at[slot])

    def start(c, slot):
        for cp in _copies(c, slot):
            cp.start()

    def wait(c, slot):
        for cp in _copies(c, slot):
            cp.wait()

    for c in range(NBUF):
        start(c, c)

    def chunk_body(i, carry):
        slot = lax.rem(i, NBUF)
        wait(i, slot)
        for s in range(CS):
            d = obuf[slot, s] - lbuf[slot, s]
            out_ref[i * CS + s] = jnp.sum(d * d) * INV_N

        @pl.when(i + NBUF < NCH)
        def _():
            start(i + NBUF, slot)

        return carry

    lax.fori_loop(0, NCH, chunk_body, 0)


def _tc_losses(output, label):
    return pl.pallas_call(
        _mse_manual_body,
        in_specs=[
            pl.BlockSpec(memory_space=pl.ANY),
            pl.BlockSpec(memory_space=pl.ANY),
        ],
        out_specs=pl.BlockSpec(memory_space=pltpu.SMEM),
        out_shape=jax.ShapeDtypeStruct((S_TC,), jnp.float32),
        scratch_shapes=[
            pltpu.VMEM((NBUF, CS, R, C), jnp.float32),
            pltpu.VMEM((NBUF, CS, R, C), jnp.float32),
            pltpu.SemaphoreType.DMA((NBUF,)),
            pltpu.SemaphoreType.DMA((NBUF,)),
        ],
    )(output, label)


RCH = 128
NCHK = R // RCH
SC_RING = 3


def _sc_partial_body(o_hbm, l_hbm, out_hbm, obuf, lbuf, res_v, semA, semB, semC):
    cid = lax.axis_index("c")
    sid = lax.axis_index("s")
    t = sid * 2 + cid
    sems = (semA, semB, semC)
    lane = lax.iota(jnp.int32, L)

    def issue(ch, slot):
        ho = pltpu.async_copy(
            o_hbm.at[S_TC + t, pl.ds(ch * RCH, RCH), :], obuf.at[slot],
            sems[slot])
        hl = pltpu.async_copy(
            l_hbm.at[S_TC + t, pl.ds(ch * RCH, RCH), :], lbuf.at[slot],
            sems[slot])
        return ho, hl

    pending = {0: issue(0, 0), 1: issue(1, 1)}
    accs = tuple(jnp.zeros((L,), jnp.float32) for _ in range(C // L))
    for ch in range(NCHK):
        slot = ch % SC_RING
        if ch + 2 < NCHK:
            pending[ch + 2] = issue(ch + 2, (ch + 2) % SC_RING)
        ho, hl = pending.pop(ch)
        ho.wait()
        hl.wait()

        def row_body(r, a):
            new = []
            for j in range(C // L):
                x = obuf[slot, r, pl.ds(j * L, L)] - lbuf[slot, r, pl.ds(j * L, L)]
                new.append(a[j] + x * x)
            return tuple(new)

        accs = lax.fori_loop(0, RCH, row_body, accs, unroll=2)

    tot = accs[0]
    for j in range(1, C // L):
        tot = tot + accs[j]
    ssum = jnp.sum(tot)
    res_v[pl.ds(0, L)] = jnp.zeros((L,), jnp.float32) + ssum
    pltpu.sync_copy(res_v.at[pl.ds(0, 8)], out_hbm.at[pl.ds(t * 8, 8)])


if K_SC > 0:
    _sc_partial = functools.partial(
        pl.kernel,
        out_type=jax.ShapeDtypeStruct((NT * 8,), jnp.float32),
        mesh=plsc.VectorSubcoreMesh(core_axis_name="c", subcore_axis_name="s"),
        compiler_params=pltpu.CompilerParams(needs_layout_passes=False),
        scratch_types=[
            pltpu.VMEM((SC_RING, RCH, C), jnp.float32),
            pltpu.VMEM((SC_RING, RCH, C), jnp.float32),
            pltpu.VMEM((L,), jnp.float32),
            pltpu.SemaphoreType.DMA,
            pltpu.SemaphoreType.DMA,
            pltpu.SemaphoreType.DMA,
        ],
    )(_sc_partial_body)


def _topk_body(v, out_hbm, out_v):
    lane = lax.iota(jnp.int32, L)
    nv = B // L
    idx = [lane + j * L for j in range(nv)]
    big = jnp.int32(2 ** 30)
    outs = [jnp.zeros((L,), jnp.float32) for _ in range(TOPK_K // L)]
    for r in range(TOPK_K):
        t = v[0]
        for j in range(1, nv):
            t = jnp.maximum(t, v[j])
        m = jnp.max(t)
        c = jnp.where(v[0] == m, idx[0], big)
        for j in range(1, nv):
            c = jnp.minimum(c, jnp.where(v[j] == m, idx[j], big))
        mi = jnp.min(c)
        for j in range(nv):
            v[j] = jnp.where(idx[j] == mi, jnp.float32(-1.0), v[j])
        q, p = divmod(r, L)
        outs[q] = jnp.where(lane == p, m, outs[q])
    for q in range(TOPK_K // L):
        out_v[pl.ds(q * L, L)] = outs[q]
    pltpu.sync_copy(out_v, out_hbm)


if K_SC > 0:
    @functools.partial(
        pl.kernel,
        out_type=jax.ShapeDtypeStruct((TOPK_K,), jnp.float32),
        mesh=plsc.VectorSubcoreMesh(core_axis_name="c", subcore_axis_name="s"),
        compiler_params=pltpu.CompilerParams(needs_layout_passes=False),
        scratch_types=[
            pltpu.VMEM((S_TC,), jnp.float32),
            pltpu.VMEM((NT * 8,), jnp.float32),
            pltpu.VMEM((TOPK_K,), jnp.float32),
        ],
    )
    def _topk_sc(tc_hbm, part_hbm, out_hbm, tc_v, part_v, out_v):
        cid = lax.axis_index("c")
        sid = lax.axis_index("s")

        @pl.when(jnp.logical_and(cid == 0, sid == 0))
        def _():
            pltpu.sync_copy(tc_hbm, tc_v)
            pltpu.sync_copy(part_hbm, part_v)
            lane = lax.iota(jnp.int32, L)
            v = [tc_v[pl.ds(j * L, L)] for j in range(S_TC // L)]
            for q in range(K_SC // L):
                vals = plsc.load_gather(part_v, [(lane + q * L) * 8])
                v.append(vals * INV_N)
            _topk_body(v, out_hbm, out_v)

    def kernel(output, label):
        tc = _tc_losses(output, label)
        part = _sc_partial(output, label)
        return _topk_sc(tc, part)
else:
    @functools.partial(
        pl.kernel,
        out_type=jax.ShapeDtypeStruct((TOPK_K,), jnp.float32),
        mesh=plsc.VectorSubcoreMesh(core_axis_name="c", subcore_axis_name="s"),
        compiler_params=pltpu.CompilerParams(needs_layout_passes=False),
        scratch_types=[
            pltpu.VMEM((B,), jnp.float32),
            pltpu.VMEM((TOPK_K,), jnp.float32),
        ],
    )
    def _topk_sc(tc_hbm, out_hbm, tc_v, out_v):
        cid = lax.axis_index("c")
        sid = lax.axis_index("s")

        @pl.when(jnp.logical_and(cid == 0, sid == 0))
        def _():
            pltpu.sync_copy(tc_hbm, tc_v)
            v = [tc_v[pl.ds(j * L, L)] for j in range(B // L)]
            _topk_body(v, out_hbm, out_v)

    def kernel(output, label):
        return _topk_sc(_tc_losses(output, label))


# speedup vs baseline: 1.0573x; 1.0573x over previous
import functools

import jax
import jax.numpy as jnp
from jax import lax
from jax.experimental import pallas as pl
from jax.experimental.pallas import tpu as pltpu
from jax.experimental.pallas import tpu_sc as plsc

B = 128
R = 2048
C = 128
TOPK_K = 32
INV_N = 1.0 / (R * C)

S_TC = 128
K_SC = B - S_TC
BB = 4

NT = 32
RT = R // NT
L = 16


NBUF = 6


def _mse_manual_body(o_hbm, l_hbm, out_ref, obuf, lbuf, osem, lsem):
    H = R // 2

    def start(c, slot):
        for arr, buf, sem in ((o_hbm, obuf, osem), (l_hbm, lbuf, lsem)):
            pltpu.async_copy(
                arr.at[c, pl.ds(0, H), :], buf.at[slot, pl.ds(0, H), :],
                sem.at[slot], priority=1)
            pltpu.async_copy(
                arr.at[c, pl.ds(H, H), :], buf.at[slot, pl.ds(H, H), :],
                sem.at[slot], priority=1)

    def wait(c, slot):
        for arr, buf, sem in ((o_hbm, obuf, osem), (l_hbm, lbuf, lsem)):
            pltpu.make_async_copy(
                arr.at[c, pl.ds(0, H), :], buf.at[slot, pl.ds(0, H), :],
                sem.at[slot]).wait()
            pltpu.make_async_copy(
                arr.at[c, pl.ds(H, H), :], buf.at[slot, pl.ds(H, H), :],
                sem.at[slot]).wait()

    for c in range(NBUF):
        start(c, c)

    def chunk_body(i, carry):
        slot = lax.rem(i, NBUF)
        wait(i, slot)
        d = obuf[slot] - lbuf[slot]
        out_ref[i] = jnp.sum(d * d) * INV_N

        @pl.when(i + NBUF < S_TC)
        def _():
            start(i + NBUF, slot)

        return carry

    lax.fori_loop(0, S_TC, chunk_body, 0)


def _tc_losses(output, label):
    return pl.pallas_call(
        _mse_manual_body,
        in_specs=[
            pl.BlockSpec(memory_space=pl.ANY),
            pl.BlockSpec(memory_space=pl.ANY),
        ],
        out_specs=pl.BlockSpec(memory_space=pltpu.SMEM),
        out_shape=jax.ShapeDtypeStruct((S_TC,), jnp.float32),
        scratch_shapes=[
            pltpu.VMEM((NBUF, R, C), jnp.float32),
            pltpu.VMEM((NBUF, R, C), jnp.float32),
            pltpu.SemaphoreType.DMA((NBUF,)),
            pltpu.SemaphoreType.DMA((NBUF,)),
        ],
    )(output, label)


def _sc_partial_body(o_hbm, l_hbm, out_hbm, obuf, lbuf, res_v, sem0, sem1):
    cid = lax.axis_index("c")
    sid = lax.axis_index("s")
    t = sid * 2 + cid
    r0 = t * RT
    sems = (sem0, sem1)
    lane = lax.iota(jnp.int32, L)

    def issue(s, slot):
        ho = pltpu.async_copy(
            o_hbm.at[S_TC + s, pl.ds(r0, RT), :], obuf.at[slot], sems[slot])
        hl = pltpu.async_copy(
            l_hbm.at[S_TC + s, pl.ds(r0, RT), :], lbuf.at[slot], sems[slot])
        return ho, hl

    pending = {0: issue(0, 0)}
    group = [jnp.zeros((L,), jnp.float32) for _ in range(K_SC // L)]
    for s in range(K_SC):
        slot = s % 2
        if s + 1 < K_SC:
            pending[s + 1] = issue(s + 1, (s + 1) % 2)
        ho, hl = pending.pop(s)
        ho.wait()
        hl.wait()

        def row_body(r, accs):
            new = []
            for j in range(C // L):
                o = obuf[slot, r, pl.ds(j * L, L)]
                x = o - lbuf[slot, r, pl.ds(j * L, L)]
                new.append(accs[j] + x * x)
            return tuple(new)

        accs = lax.fori_loop(
            0, RT, row_body,
            tuple(jnp.zeros((L,), jnp.float32) for _ in range(C // L)),
            unroll=2)
        tot = accs[0]
        for j in range(1, C // L):
            tot = tot + accs[j]
        ssum = jnp.sum(tot)
        q, p = divmod(s, L)
        group[q] = jnp.where(lane == p, ssum, group[q])

    for q in range(K_SC // L):
        res_v[pl.ds(q * L, L)] = group[q]
    pltpu.sync_copy(res_v, out_hbm.at[t])


if K_SC > 0:
    _sc_partial = functools.partial(
        pl.kernel,
        out_type=jax.ShapeDtypeStruct((NT, K_SC), jnp.float32),
        mesh=plsc.VectorSubcoreMesh(core_axis_name="c", subcore_axis_name="s"),
        compiler_params=pltpu.CompilerParams(needs_layout_passes=False),
        scratch_types=[
            pltpu.VMEM((2, RT, C), jnp.float32),
            pltpu.VMEM((2, RT, C), jnp.float32),
            pltpu.VMEM((K_SC,), jnp.float32),
            pltpu.SemaphoreType.DMA,
            pltpu.SemaphoreType.DMA,
        ],
    )(_sc_partial_body)


def _topk_body(v, out_hbm, out_v):
    lane = lax.iota(jnp.int32, L)
    nv = B // L
    idx = [lane + j * L for j in range(nv)]
    big = jnp.int32(2 ** 30)
    outs = [jnp.zeros((L,), jnp.float32) for _ in range(TOPK_K // L)]
    for r in range(TOPK_K):
        t = v[0]
        for j in range(1, nv):
            t = jnp.maximum(t, v[j])
        m = jnp.max(t)
        c = jnp.where(v[0] == m, idx[0], big)
        for j in range(1, nv):
            c = jnp.minimum(c, jnp.where(v[j] == m, idx[j], big))
        mi = jnp.min(c)
        for j in range(nv):
            v[j] = jnp.where(idx[j] == mi, jnp.float32(-1.0), v[j])
        q, p = divmod(r, L)
        outs[q] = jnp.where(lane == p, m, outs[q])
    for q in range(TOPK_K // L):
        out_v[pl.ds(q * L, L)] = outs[q]
    pltpu.sync_copy(out_v, out_hbm)


if K_SC > 0:
    @functools.partial(
        pl.kernel,
        out_type=jax.ShapeDtypeStruct((TOPK_K,), jnp.float32),
        mesh=plsc.VectorSubcoreMesh(core_axis_name="c", subcore_axis_name="s"),
        compiler_params=pltpu.CompilerParams(needs_layout_passes=False),
        scratch_types=[
            pltpu.VMEM((S_TC,), jnp.float32),
            pltpu.VMEM((NT, K_SC), jnp.float32),
            pltpu.VMEM((TOPK_K,), jnp.float32),
        ],
    )
    def _topk_sc(tc_hbm, part_hbm, out_hbm, tc_v, part_v, out_v):
        cid = lax.axis_index("c")
        sid = lax.axis_index("s")

        @pl.when(jnp.logical_and(cid == 0, sid == 0))
        def _():
            pltpu.sync_copy(tc_hbm, tc_v)
            pltpu.sync_copy(part_hbm, part_v)
            v = [tc_v[pl.ds(j * L, L)] for j in range(S_TC // L)]
            for q in range(K_SC // L):
                acc = part_v[0, pl.ds(q * L, L)]
                for t in range(1, NT):
                    acc = acc + part_v[t, pl.ds(q * L, L)]
                v.append(acc * INV_N)
            _topk_body(v, out_hbm, out_v)

    def kernel(output, label):
        tc = _tc_losses(output, label)
        part = _sc_partial(output, label)
        return _topk_sc(tc, part)
else:
    @functools.partial(
        pl.kernel,
        out_type=jax.ShapeDtypeStruct((TOPK_K,), jnp.float32),
        mesh=plsc.VectorSubcoreMesh(core_axis_name="c", subcore_axis_name="s"),
        compiler_params=pltpu.CompilerParams(needs_layout_passes=False),
        scratch_types=[
            pltpu.VMEM((B,), jnp.float32),
            pltpu.VMEM((TOPK_K,), jnp.float32),
        ],
    )
    def _topk_sc(tc_hbm, out_hbm, tc_v, out_v):
        cid = lax.axis_index("c")
        sid = lax.axis_index("s")

        @pl.when(jnp.logical_and(cid == 0, sid == 0))
        def _():
            pltpu.sync_copy(tc_hbm, tc_v)
            v = [tc_v[pl.ds(j * L, L)] for j in range(B // L)]
            _topk_body(v, out_hbm, out_v)

    def kernel(output, label):
        return _topk_sc(_tc_losses(output, label))


# speedup vs baseline: 1.0811x; 1.0225x over previous
import functools

import jax
import jax.numpy as jnp
from jax import lax
from jax.experimental import pallas as pl
from jax.experimental.pallas import tpu as pltpu
from jax.experimental.pallas import tpu_sc as plsc

B = 128
R = 2048
C = 128
TOPK_K = 32
INV_N = 1.0 / (R * C)

S_TC = 128
K_SC = B - S_TC
BB = 4

NT = 32
RT = R // NT
L = 16


NBUF = 6


def _mse_manual_body(o_hbm, l_hbm, out_ref, obuf, lbuf, osem, lsem):
    H = R // 2

    def start(c, slot):
        pltpu.make_async_copy(
            o_hbm.at[c, pl.ds(0, H), :], obuf.at[slot, pl.ds(0, H), :],
            osem.at[slot]).start()
        pltpu.make_async_copy(
            l_hbm.at[c, pl.ds(H, H), :], lbuf.at[slot, pl.ds(H, H), :],
            lsem.at[slot]).start()
        pltpu.make_async_copy(
            o_hbm.at[c, pl.ds(H, H), :], obuf.at[slot, pl.ds(H, H), :],
            osem.at[slot]).start()
        pltpu.make_async_copy(
            l_hbm.at[c, pl.ds(0, H), :], lbuf.at[slot, pl.ds(0, H), :],
            lsem.at[slot]).start()

    def wait(c, slot):
        for arr, buf, sem in ((o_hbm, obuf, osem), (l_hbm, lbuf, lsem)):
            pltpu.make_async_copy(
                arr.at[c, pl.ds(0, H), :], buf.at[slot, pl.ds(0, H), :],
                sem.at[slot]).wait()
            pltpu.make_async_copy(
                arr.at[c, pl.ds(H, H), :], buf.at[slot, pl.ds(H, H), :],
                sem.at[slot]).wait()

    for c in range(NBUF):
        start(c, c)

    def chunk_body(i, carry):
        slot = lax.rem(i, NBUF)
        wait(i, slot)
        d = obuf[slot] - lbuf[slot]
        out_ref[i] = jnp.sum(d * d) * INV_N

        @pl.when(i + NBUF < S_TC)
        def _():
            start(i + NBUF, slot)

        return carry

    lax.fori_loop(0, S_TC, chunk_body, 0)


def _tc_losses(output, label):
    return pl.pallas_call(
        _mse_manual_body,
        in_specs=[
            pl.BlockSpec(memory_space=pl.ANY),
            pl.BlockSpec(memory_space=pl.ANY),
        ],
        out_specs=pl.BlockSpec(memory_space=pltpu.SMEM),
        out_shape=jax.ShapeDtypeStruct((S_TC,), jnp.float32),
        scratch_shapes=[
            pltpu.VMEM((NBUF, R, C), jnp.float32),
            pltpu.VMEM((NBUF, R, C), jnp.float32),
            pltpu.SemaphoreType.DMA((NBUF,)),
            pltpu.SemaphoreType.DMA((NBUF,)),
        ],
    )(output, label)


def _sc_partial_body(o_hbm, l_hbm, out_hbm, obuf, lbuf, res_v, sem0, sem1):
    cid = lax.axis_index("c")
    sid = lax.axis_index("s")
    t = sid * 2 + cid
    r0 = t * RT
    sems = (sem0, sem1)
    lane = lax.iota(jnp.int32, L)

    def issue(s, slot):
        ho = pltpu.async_copy(
            o_hbm.at[S_TC + s, pl.ds(r0, RT), :], obuf.at[slot], sems[slot])
        hl = pltpu.async_copy(
            l_hbm.at[S_TC + s, pl.ds(r0, RT), :], lbuf.at[slot], sems[slot])
        return ho, hl

    pending = {0: issue(0, 0)}
    group = [jnp.zeros((L,), jnp.float32) for _ in range(K_SC // L)]
    for s in range(K_SC):
        slot = s % 2
        if s + 1 < K_SC:
            pending[s + 1] = issue(s + 1, (s + 1) % 2)
        ho, hl = pending.pop(s)
        ho.wait()
        hl.wait()

        def row_body(r, accs):
            new = []
            for j in range(C // L):
                o = obuf[slot, r, pl.ds(j * L, L)]
                x = o - lbuf[slot, r, pl.ds(j * L, L)]
                new.append(accs[j] + x * x)
            return tuple(new)

        accs = lax.fori_loop(
            0, RT, row_body,
            tuple(jnp.zeros((L,), jnp.float32) for _ in range(C // L)),
            unroll=2)
        tot = accs[0]
        for j in range(1, C // L):
            tot = tot + accs[j]
        ssum = jnp.sum(tot)
        q, p = divmod(s, L)
        group[q] = jnp.where(lane == p, ssum, group[q])

    for q in range(K_SC // L):
        res_v[pl.ds(q * L, L)] = group[q]
    pltpu.sync_copy(res_v, out_hbm.at[t])


if K_SC > 0:
    _sc_partial = functools.partial(
        pl.kernel,
        out_type=jax.ShapeDtypeStruct((NT, K_SC), jnp.float32),
        mesh=plsc.VectorSubcoreMesh(core_axis_name="c", subcore_axis_name="s"),
        compiler_params=pltpu.CompilerParams(needs_layout_passes=False),
        scratch_types=[
            pltpu.VMEM((2, RT, C), jnp.float32),
            pltpu.VMEM((2, RT, C), jnp.float32),
            pltpu.VMEM((K_SC,), jnp.float32),
            pltpu.SemaphoreType.DMA,
            pltpu.SemaphoreType.DMA,
        ],
    )(_sc_partial_body)


def _topk_body(v, out_hbm, out_v):
    lane = lax.iota(jnp.int32, L)
    nv = B // L
    idx = [lane + j * L for j in range(nv)]
    big = jnp.int32(2 ** 30)
    outs = [jnp.zeros((L,), jnp.float32) for _ in range(TOPK_K // L)]
    for r in range(TOPK_K):
        t = v[0]
        for j in range(1, nv):
            t = jnp.maximum(t, v[j])
        m = jnp.max(t)
        c = jnp.where(v[0] == m, idx[0], big)
        for j in range(1, nv):
            c = jnp.minimum(c, jnp.where(v[j] == m, idx[j], big))
        mi = jnp.min(c)
        for j in range(nv):
            v[j] = jnp.where(idx[j] == mi, jnp.float32(-1.0), v[j])
        q, p = divmod(r, L)
        outs[q] = jnp.where(lane == p, m, outs[q])
    for q in range(TOPK_K // L):
        out_v[pl.ds(q * L, L)] = outs[q]
    pltpu.sync_copy(out_v, out_hbm)


if K_SC > 0:
    @functools.partial(
        pl.kernel,
        out_type=jax.ShapeDtypeStruct((TOPK_K,), jnp.float32),
        mesh=plsc.VectorSubcoreMesh(core_axis_name="c", subcore_axis_name="s"),
        compiler_params=pltpu.CompilerParams(needs_layout_passes=False),
        scratch_types=[
            pltpu.VMEM((S_TC,), jnp.float32),
            pltpu.VMEM((NT, K_SC), jnp.float32),
            pltpu.VMEM((TOPK_K,), jnp.float32),
        ],
    )
    def _topk_sc(tc_hbm, part_hbm, out_hbm, tc_v, part_v, out_v):
        cid = lax.axis_index("c")
        sid = lax.axis_index("s")

        @pl.when(jnp.logical_and(cid == 0, sid == 0))
        def _():
            pltpu.sync_copy(tc_hbm, tc_v)
            pltpu.sync_copy(part_hbm, part_v)
            v = [tc_v[pl.ds(j * L, L)] for j in range(S_TC // L)]
            for q in range(K_SC // L):
                acc = part_v[0, pl.ds(q * L, L)]
                for t in range(1, NT):
                    acc = acc + part_v[t, pl.ds(q * L, L)]
                v.append(acc * INV_N)
            _topk_body(v, out_hbm, out_v)

    def kernel(output, label):
        tc = _tc_losses(output, label)
        part = _sc_partial(output, label)
        return _topk_sc(tc, part)
else:
    @functools.partial(
        pl.kernel,
        out_type=jax.ShapeDtypeStruct((TOPK_K,), jnp.float32),
        mesh=plsc.VectorSubcoreMesh(core_axis_name="c", subcore_axis_name="s"),
        compiler_params=pltpu.CompilerParams(needs_layout_passes=False),
        scratch_types=[
            pltpu.VMEM((B,), jnp.float32),
            pltpu.VMEM((TOPK_K,), jnp.float32),
        ],
    )
    def _topk_sc(tc_hbm, out_hbm, tc_v, out_v):
        cid = lax.axis_index("c")
        sid = lax.axis_index("s")

        @pl.when(jnp.logical_and(cid == 0, sid == 0))
        def _():
            pltpu.sync_copy(tc_hbm, tc_v)
            v = [tc_v[pl.ds(j * L, L)] for j in range(B // L)]
            _topk_body(v, out_hbm, out_v)

    def kernel(output, label):
        return _topk_sc(_tc_losses(output, label))
